# baseline (speedup 1.0000x reference)
import functools

import jax
import jax.numpy as jnp
from jax import lax
from jax.experimental import pallas as pl
from jax.experimental.pallas import tpu as pltpu
from jax.experimental.pallas import tpu_sc as plsc

_LANES = 16
_NUM_CORES = 2
_NUM_SUBCORES = 16
_NUM_WORKERS = _NUM_CORES * _NUM_SUBCORES


def _mean_body(att_ref, avgT_ref):
    att = att_ref[0]
    avg = jnp.sum(att, axis=0) * (1.0 / att.shape[0])
    avgT_ref[0] = jnp.swapaxes(avg, 0, 1)


def _head_mean_t(attention_weights):
    B, H, N, _ = attention_weights.shape
    BI = 128
    return pl.pallas_call(
        _mean_body,
        grid=(B, N // BI),
        in_specs=[pl.BlockSpec((1, H, BI, N), lambda b, i: (b, 0, i, 0))],
        out_specs=pl.BlockSpec((1, N, BI), lambda b, i: (b, 0, i)),
        out_shape=jax.ShapeDtypeStruct((B, N, N), jnp.float32),
    )(attention_weights)


def _make_sc_bag(B, N, T):
    stripe = 128
    gps = stripe // _LANES
    n_stripes = B * N // stripe
    n_halves = _NUM_WORKERS // n_stripes
    jh_len = N // n_halves
    groups_per_b = N // _LANES
    G = B * groups_per_b
    mesh = plsc.VectorSubcoreMesh(core_axis_name="c", subcore_axis_name="s")
    unroll = 4

    @functools.partial(
        pl.kernel,
        mesh=mesh,
        out_type=jax.ShapeDtypeStruct((n_halves, G, T * _LANES), jnp.float32),
        scratch_types=[
            pltpu.VMEM((jh_len, stripe), jnp.float32),
            pltpu.VMEM((jh_len, stripe), jnp.int32),
            pltpu.VMEM((gps, T * _LANES), jnp.float32),
        ],
    )
    def sc_bag(avgT_hbm, et_hbm, s_hbm, avg_v, et_v, bins_v):
        wid = lax.axis_index("s") * _NUM_CORES + lax.axis_index("c")
        sid = wid // n_halves
        jh = wid % n_halves
        b = sid // (N // stripe)
        i0 = (sid % (N // stripe)) * stripe
        j0 = jh * jh_len
        pltpu.sync_copy(avgT_hbm.at[b, pl.ds(j0, jh_len), pl.ds(i0, stripe)], avg_v)
        pltpu.sync_copy(et_hbm.at[b, pl.ds(j0, jh_len), pl.ds(i0, stripe)], et_v)
        zero16 = jnp.zeros((_LANES,), jnp.float32)
        for gi in range(gps):
            l0 = gi * _LANES

            def j_body(jj, accs):
                accs = list(accs)
                for u in range(unroll):
                    j = jj * unroll + u
                    w = avg_v[j, pl.ds(l0, _LANES)]
                    tv = et_v[j, pl.ds(l0, _LANES)]
                    for t in range(T):
                        accs[t] = accs[t] + jnp.where(tv == t, w, 0.0)
                return tuple(accs)

            acc = lax.fori_loop(0, jh_len // unroll, j_body, (zero16,) * T)
            for t in range(T):
                bins_v[gi, pl.ds(t * _LANES, _LANES)] = acc[t]
        pltpu.sync_copy(bins_v, s_hbm.at[jh, pl.ds(sid * gps, gps)])

    return sc_bag


def _combine_body(s_ref, emb_ref, out_ref):
    S3 = s_ref[...]
    H2, G, TL = S3.shape
    T = emb_ref.shape[0]
    Sg = (S3[0] + S3[1]).reshape(G, T, _LANES)
    S = jnp.swapaxes(Sg, 1, 2).reshape(G * _LANES, T)
    out_ref[...] = jax.lax.dot_general(
        S, emb_ref[...], (((1,), (0,)), ((), ())),
        preferred_element_type=jnp.float32,
        precision=jax.lax.Precision.HIGHEST)


def _combine(s, embedding_table, R):
    H2, G, TL = s.shape
    T, D = embedding_table.shape
    return pl.pallas_call(
        _combine_body,
        in_specs=[
            pl.BlockSpec((H2, G, TL), lambda: (0, 0, 0)),
            pl.BlockSpec((T, D), lambda: (0, 0)),
        ],
        out_specs=pl.BlockSpec((R, D), lambda: (0, 0)),
        out_shape=jax.ShapeDtypeStruct((R, D), jnp.float32),
    )(s, embedding_table)


def kernel(attention_weights, edge_type_matrix, embedding_table):
    B, H, N, _ = attention_weights.shape
    T, D = embedding_table.shape
    R = B * N
    avgT = _head_mean_t(attention_weights)
    s = _make_sc_bag(B, N, T)(avgT, edge_type_matrix.astype(jnp.int32))
    out = _combine(s, embedding_table, R)
    return out.reshape(B, N, D)


# speedup vs baseline: 1.6483x; 1.6483x over previous
import functools

import jax
import jax.numpy as jnp
from jax import lax
from jax.experimental import pallas as pl
from jax.experimental.pallas import tpu as pltpu
from jax.experimental.pallas import tpu_sc as plsc

_LANES = 16
_NUM_CORES = 2
_NUM_SUBCORES = 16
_NUM_WORKERS = _NUM_CORES * _NUM_SUBCORES


def _mean_body(att_ref, avgT_ref):
    att = att_ref[0]
    avg = jnp.sum(att, axis=0) * (1.0 / att.shape[0])
    avgT_ref[0] = jnp.swapaxes(avg, 0, 1)


def _head_mean_t(attention_weights):
    B, H, N, _ = attention_weights.shape
    BI = 128
    return pl.pallas_call(
        _mean_body,
        grid=(B, N // BI),
        in_specs=[pl.BlockSpec((1, H, BI, N), lambda b, i: (b, 0, i, 0))],
        out_specs=pl.BlockSpec((1, N, BI), lambda b, i: (b, 0, i)),
        out_shape=jax.ShapeDtypeStruct((B, N, N), jnp.float32),
    )(attention_weights)


def _make_sc_bag(B, N, T):
    stripe = 128
    gps = stripe // _LANES
    n_stripes = B * N // stripe
    n_halves = _NUM_WORKERS // n_stripes
    jh_len = N // n_halves
    groups_per_b = N // _LANES
    G = B * groups_per_b
    mesh = plsc.VectorSubcoreMesh(core_axis_name="c", subcore_axis_name="s")
    unroll = 4

    @functools.partial(
        pl.kernel,
        mesh=mesh,
        out_type=jax.ShapeDtypeStruct((n_halves, G, T * _LANES), jnp.float32),
        scratch_types=[
            pltpu.VMEM((jh_len, stripe), jnp.float32),
            pltpu.VMEM((jh_len, stripe), jnp.int32),
            pltpu.VMEM((gps, T * _LANES), jnp.float32),
        ],
    )
    def sc_bag(avgT_hbm, et_hbm, s_hbm, avg_v, et_v, bins_v):
        wid = lax.axis_index("s") * _NUM_CORES + lax.axis_index("c")
        sid = wid // n_halves
        jh = wid % n_halves
        b = sid // (N // stripe)
        i0 = (sid % (N // stripe)) * stripe
        j0 = jh * jh_len
        pltpu.sync_copy(avgT_hbm.at[b, pl.ds(j0, jh_len), pl.ds(i0, stripe)], avg_v)
        pltpu.sync_copy(et_hbm.at[b, pl.ds(j0, jh_len), pl.ds(i0, stripe)], et_v)
        zero16 = jnp.zeros((_LANES,), jnp.float32)
        chunk = 32

        def group_body(gi, carry):
            l0 = pl.multiple_of(gi * _LANES, _LANES)
            for t in range(T):
                bins_v[gi, pl.ds(t * _LANES, _LANES)] = zero16

            def j_chunk(cc, c2):
                jb = cc * chunk
                acc = [zero16] * T
                for u in range(chunk):
                    w = avg_v[jb + u, pl.ds(l0, _LANES)]
                    tv = et_v[jb + u, pl.ds(l0, _LANES)]
                    for t in range(T):
                        acc[t] = acc[t] + jnp.where(tv == t, w, 0.0)
                for t in range(T):
                    sl = pl.ds(t * _LANES, _LANES)
                    bins_v[gi, sl] = bins_v[gi, sl] + acc[t]
                return c2

            lax.fori_loop(0, jh_len // chunk, j_chunk, 0)
            return carry

        lax.fori_loop(0, gps, group_body, 0)
        pltpu.sync_copy(bins_v, s_hbm.at[jh, pl.ds(sid * gps, gps)])

    return sc_bag


def _combine_body(s_ref, emb_ref, out_ref):
    S3 = s_ref[...]
    H2, G, TL = S3.shape
    T = emb_ref.shape[0]
    Sg = (S3[0] + S3[1]).reshape(G, T, _LANES)
    S = jnp.swapaxes(Sg, 1, 2).reshape(G * _LANES, T)
    out_ref[...] = jax.lax.dot_general(
        S, emb_ref[...], (((1,), (0,)), ((), ())),
        preferred_element_type=jnp.float32,
        precision=jax.lax.Precision.HIGHEST)


def _combine(s, embedding_table, R):
    H2, G, TL = s.shape
    T, D = embedding_table.shape
    return pl.pallas_call(
        _combine_body,
        in_specs=[
            pl.BlockSpec((H2, G, TL), lambda: (0, 0, 0)),
            pl.BlockSpec((T, D), lambda: (0, 0)),
        ],
        out_specs=pl.BlockSpec((R, D), lambda: (0, 0)),
        out_shape=jax.ShapeDtypeStruct((R, D), jnp.float32),
    )(s, embedding_table)


def kernel(attention_weights, edge_type_matrix, embedding_table):
    B, H, N, _ = attention_weights.shape
    T, D = embedding_table.shape
    R = B * N
    avgT = _head_mean_t(attention_weights)
    s = _make_sc_bag(B, N, T)(avgT, edge_type_matrix.astype(jnp.int32))
    out = _combine(s, embedding_table, R)
    return out.reshape(B, N, D)
